# baseline (speedup 1.0000x reference)
import jax
import jax.numpy as jnp
from jax.experimental import pallas as pl


_BLOCK = 1000


def _phi_body(src_ref, e_ref, tgt_ref, out_ref):
    ones = jnp.ones((e_ref.shape[1], src_ref.shape[1]), jnp.float32)
    s = jnp.dot(e_ref[...], ones, preferred_element_type=jnp.float32)
    gate = jax.nn.sigmoid(s * (1.0 / e_ref.shape[1]))
    out_ref[...] = src_ref[...] * gate + tgt_ref[...]


def kernel(src, e, tgt):
    n, d = src.shape
    de = e.shape[1]
    grid = n // _BLOCK
    return pl.pallas_call(
        _phi_body,
        grid=(grid,),
        in_specs=[
            pl.BlockSpec((_BLOCK, d), lambda i: (i, 0)),
            pl.BlockSpec((_BLOCK, de), lambda i: (i, 0)),
            pl.BlockSpec((_BLOCK, d), lambda i: (i, 0)),
        ],
        out_specs=pl.BlockSpec((_BLOCK, d), lambda i: (i, 0)),
        out_shape=jax.ShapeDtypeStruct((n, d), src.dtype),
    )(src, e, tgt)


# speedup vs baseline: 1.4891x; 1.4891x over previous
import jax
import jax.numpy as jnp
from jax.experimental import pallas as pl


_BLOCK = 8000


def _phi_body(src_ref, e_ref, tgt_ref, out_ref):
    ones = jnp.ones((e_ref.shape[1], src_ref.shape[1]), jnp.float32)
    s = jnp.dot(e_ref[...], ones, preferred_element_type=jnp.float32)
    gate = jax.nn.sigmoid(s * (1.0 / e_ref.shape[1]))
    out_ref[...] = src_ref[...] * gate + tgt_ref[...]


def kernel(src, e, tgt):
    n, d = src.shape
    de = e.shape[1]
    grid = n // _BLOCK
    return pl.pallas_call(
        _phi_body,
        grid=(grid,),
        in_specs=[
            pl.BlockSpec((_BLOCK, d), lambda i: (i, 0)),
            pl.BlockSpec((_BLOCK, de), lambda i: (i, 0)),
            pl.BlockSpec((_BLOCK, d), lambda i: (i, 0)),
        ],
        out_specs=pl.BlockSpec((_BLOCK, d), lambda i: (i, 0)),
        out_shape=jax.ShapeDtypeStruct((n, d), src.dtype),
    )(src, e, tgt)
